# baseline (speedup 1.0000x reference)
import dataclasses
import functools

import jax
import jax.numpy as jnp
from jax import lax
from jax.experimental import pallas as pl
from jax.experimental.pallas import tpu as pltpu
from jax.experimental.pallas import tpu_sc as plsc

GX, GY = 432, 496
NCELL = GX * GY
MAXV, MAXP, C = 20000, 32, 4
N = 200000
NT = 16
CPT = NCELL // NT
NCV = CPT // 16
CAP = 8192

VOX_WORDS = 2560512
VOXDUMPW = 2560480
COOR_WORDS = 60160
NPV_WORDS = 20096
CN_ROWS = 20688

VS0, VS1, VS2 = 0.16, 0.16, 4.0
PR0, PR1, PR2 = 0.0, -39.68, -3.0
CNTMASK = 0x3FFFF

_MESH = plsc.VectorSubcoreMesh(core_axis_name="c", subcore_axis_name="s",
                               num_cores=1)
_CP = pltpu.CompilerParams()
if "needs_layout_passes" in pltpu.CompilerParams.__dataclass_fields__:
    _CP = dataclasses.replace(_CP, needs_layout_passes=False)

NFULL, TAILP = 97, 1344


def _floor_div(q):
    ti = q.astype(jnp.int32)
    return ti - (ti.astype(jnp.float32) > q).astype(jnp.int32)


@functools.partial(
    pl.kernel,
    out_type=[
        jax.ShapeDtypeStruct((VOX_WORDS,), jnp.float32),
        jax.ShapeDtypeStruct((COOR_WORDS,), jnp.int32),
        jax.ShapeDtypeStruct((NPV_WORDS,), jnp.int32),
        jax.ShapeDtypeStruct((16,), jnp.int32),
    ],
    mesh=_MESH,
    compiler_params=_CP,
    scratch_types=[
        pltpu.HBM((CN_ROWS * 8,), jnp.int32),
        pltpu.HBM((256,), jnp.int32),
        pltpu.HBM((200704,), jnp.int32),
        pltpu.VMEM((8192,), jnp.float32),
        pltpu.VMEM((8192,), jnp.float32),
        pltpu.VMEM((CPT,), jnp.int32),
        pltpu.VMEM((CAP + 16,), jnp.int32),
        pltpu.VMEM((CAP + 16,), jnp.float32),
        pltpu.VMEM((CAP + 16,), jnp.float32),
        pltpu.VMEM((CAP + 16,), jnp.float32),
        pltpu.VMEM((CAP + 16,), jnp.float32),
        pltpu.VMEM((CAP + 16,), jnp.int32),
        pltpu.VMEM((32768,), jnp.float32),
        pltpu.VMEM((2176,), jnp.int32),
        pltpu.VMEM((2048,), jnp.int32),
        pltpu.VMEM((8192,), jnp.int32),
        pltpu.VMEM((2048,), jnp.float32),
        pltpu.VMEM((16,), jnp.int32),
        pltpu.VMEM((16,), jnp.int32),
        pltpu.VMEM((16,), jnp.float32),
        pltpu.SemaphoreType.DMA,
    ],
)
def _vox_kernel(pts_hbm, vox_out, coor_out, npv_out, vnum_out,
                cn_hbm, occ_hbm, lin_hbm,
                pbuf, pbuf2, hcell, ks_s, ks_x, ks_y, ks_z, ks_w, ks_i,
                wstage, cnst, ostage, rbuf, zf, b16, ovi, ovv, sem):
    t = lax.axis_index("s")
    lane = lax.iota(jnp.int32, 16)
    ones = jnp.ones((16,), jnp.int32)
    zeros16 = jnp.zeros((16,), jnp.int32)
    zf16 = jnp.zeros((16,), jnp.float32)
    lo = t * CPT

    def lin_of(j):
        idx = (j * 16 + lane) * 4
        x = plsc.load_gather(pbuf, [idx])
        y = plsc.load_gather(pbuf, [idx + 1])
        z = plsc.load_gather(pbuf, [idx + 2])
        cx = _floor_div((x - PR0) / jnp.float32(VS0))
        cy = _floor_div((y - PR1) / jnp.float32(VS1))
        cz = _floor_div((z - PR2) / jnp.float32(VS2))
        valid = ((cx >= 0) & (cx < GX) & (cy >= 0) & (cy < GY) & (cz == 0))
        return jnp.where(valid, cy * GX + cx, NCELL), x, y, z, idx

    def hz(i, _):
        hcell[pl.ds(i * 16, 16)] = zeros16
        return 0

    lax.fori_loop(0, NCV, hz, 0)

    p0s = pl.multiple_of(t * 12512, 8)

    def p0_chunk(c0, nvec):
        def body(j, _):
            lin, _x, _y, _z, _i = lin_of(j)
            ostage[pl.ds(j * 16, 16)] = lin
            return 0

        lax.fori_loop(0, nvec, body, 0)

    def p0(c0, _):
        pltpu.sync_copy(pts_hbm.at[pl.ds(pl.multiple_of((p0s + c0 * 2048) * 4,
                                                        8), 8192)], pbuf)
        p0_chunk(c0, 128)
        pltpu.sync_copy(ostage,
                        lin_hbm.at[pl.ds(pl.multiple_of(p0s + c0 * 2048, 8),
                                         2048)])
        return 0

    lax.fori_loop(0, 6, p0, 0)

    @pl.when(t < 15)
    def _():
        pltpu.sync_copy(pts_hbm.at[pl.ds(pl.multiple_of((p0s + 12288) * 4, 8),
                                         896)], pbuf.at[pl.ds(0, 896)])
        p0_chunk(6, 14)
        pltpu.sync_copy(ostage.at[pl.ds(0, 224)],
                        lin_hbm.at[pl.ds(pl.multiple_of(p0s + 12288, 8), 224)])

    @pl.when(t == 15)
    def _():
        pltpu.sync_copy(pts_hbm.at[pl.ds(pl.multiple_of((p0s + 12288) * 4, 8),
                                         128)], pbuf.at[pl.ds(0, 128)])
        p0_chunk(6, 2)
        pltpu.sync_copy(ostage.at[pl.ds(0, 32)],
                        lin_hbm.at[pl.ds(pl.multiple_of(p0s + 12288, 8), 32)])

    plsc.subcore_barrier()

    def p1_chunk(nvec, boff):
        def body(j, _):
            lin = rbuf[pl.ds(boff + j * 16, 16)]
            inr = (lin >= lo) & (lin < lo + CPT)

            @pl.when(jnp.sum(inr.astype(jnp.int32)) > 0)
            def _():
                cell = jnp.where(inr, lin - lo, 0)
                plsc.addupdate_scatter(hcell, [cell], ones, mask=inr)

            return 0

        lax.fori_loop(0, nvec, body, 0)

    def lin_cp(c0, half):
        return pltpu.async_copy(
            lin_hbm.at[pl.ds(pl.multiple_of(c0 * 4096, 8), 4096)],
            rbuf.at[pl.ds(half * 4096, 4096)], sem)

    lin_cp(0, 0)
    lin_cp(1, 1)

    def p1(k, _):
        for h in range(2):
            c = 2 * k + h
            pltpu.make_async_copy(
                lin_hbm.at[pl.ds(pl.multiple_of(c * 4096, 8), 4096)],
                rbuf.at[pl.ds(h * 4096, 4096)], sem).wait()
            p1_chunk(256, h * 4096)

            @pl.when(c + 2 < 48)
            def _(c=c, h=h):
                lin_cp(c + 2, h)

        return 0

    lax.fori_loop(0, 24, p1, 0)
    pltpu.sync_copy(lin_hbm.at[pl.ds(48 * 4096, 3392)],
                    rbuf.at[pl.ds(0, 3392)])
    p1_chunk(212, 0)

    def p2(i, carry):
        h = hcell[pl.ds(i * 16, 16)]
        occ = (h > 0).astype(jnp.int32)
        excl = plsc.cumsum(occ) - occ + carry
        hcell[pl.ds(i * 16, 16)] = excl << 18
        return carry + jnp.sum(occ)

    occ_t = lax.fori_loop(0, NCV, p2, jnp.int32(0))
    b16[...] = jnp.full((16,), occ_t, jnp.int32)
    pltpu.sync_copy(b16, occ_hbm.at[pl.ds(pl.multiple_of(t * 16, 8), 16)])

    def zb(i, _):
        zf[pl.ds(i * 16, 16)] = zf16
        return 0

    lax.fori_loop(0, 128, zb, 0)
    vz = pl.multiple_of(t * 160032, 8)
    for k in range(78):
        pltpu.sync_copy(zf, vox_out.at[pl.ds(vz + k * 2048, 2048)])
    pltpu.sync_copy(zf.at[pl.ds(0, 288)],
                    vox_out.at[pl.ds(vz + 78 * 2048, 288)])

    plsc.subcore_barrier()
    pltpu.sync_copy(occ_hbm, cnst.at[pl.ds(0, 256)])
    occv = plsc.load_gather(cnst, [lane * 16])
    rb = jnp.sum(jnp.where(lane < t, occv, 0))
    total_occ = jnp.sum(occv)
    vn = jnp.minimum(total_occ, MAXV)
    nout = jnp.clip(jnp.minimum(occ_t, MAXV - rb), 0, MAXV)

    @pl.when(t == 0)
    def _():
        b16[...] = jnp.full((16,), vn, jnp.int32)
        pltpu.sync_copy(b16, vnum_out)

    def emit_chunk(nvec, kc0, append, pb, loff):
        def body(j, kc):
            lin = rbuf[pl.ds(loff + j * 16, 16)]
            inr = (lin >= lo) & (lin < lo + CPT)

            def heavy(kc):
                idx = (j * 16 + lane) * 4
                x = plsc.load_gather(pb, [idx])
                y = plsc.load_gather(pb, [idx + 1])
                z = plsc.load_gather(pb, [idx + 2])
                w = plsc.load_gather(pb, [idx + 3])
                cell = jnp.where(inr, lin - lo, 0)
                h = plsc.load_gather(hcell, [cell], mask=inr)
                prior, _u = plsc.scan_count(cell, mask=inr)
                pos = (h & CNTMASK) + prior - 1
                lr = lax.shift_right_logical(h, 18)
                plsc.addupdate_scatter(hcell, [cell], ones, mask=inr)
                keep = inr & (pos < MAXP) & (lr < nout)
                rel = lr * MAXP + pos
                return append(kc, keep, rel, x, y, z, w)

            return lax.cond(jnp.sum(inr.astype(jnp.int32)) > 0,
                            heavy, lambda kc: kc, kc)

        return lax.fori_loop(0, nvec, body, kc0)

    def emit_pass(kc0, append):
        pbufs = (pbuf, pbuf2)

        def pt_cp(c0, h):
            pltpu.async_copy(
                pts_hbm.at[pl.ds(pl.multiple_of(c0 * 8192, 8), 8192)],
                pbufs[h], sem)
            pltpu.async_copy(
                lin_hbm.at[pl.ds(pl.multiple_of(c0 * 2048, 8), 2048)],
                rbuf.at[pl.ds(h * 2048, 2048)], sem)

        def pt_wait(c0, h):
            pltpu.make_async_copy(
                pts_hbm.at[pl.ds(pl.multiple_of(c0 * 8192, 8), 8192)],
                pbufs[h], sem).wait()
            pltpu.make_async_copy(
                lin_hbm.at[pl.ds(pl.multiple_of(c0 * 2048, 8), 2048)],
                rbuf.at[pl.ds(h * 2048, 2048)], sem).wait()

        pt_cp(0, 0)
        pt_cp(1, 1)

        def pc(k, kc):
            for h in range(2):
                c = 2 * k + h
                pt_wait(c, h)
                kc = emit_chunk(128, kc, append, pbufs[h], h * 2048)

                @pl.when(c + 2 < 96)
                def _(c=c, h=h):
                    pt_cp(c + 2, h)

            return kc

        kc = lax.fori_loop(0, 48, pc, kc0)
        pltpu.sync_copy(pts_hbm.at[pl.ds(96 * 8192, 8192)], pbuf)
        pltpu.sync_copy(lin_hbm.at[pl.ds(96 * 2048, 2048)],
                        rbuf.at[pl.ds(0, 2048)])
        kc = emit_chunk(128, kc, append, pbuf, 0)
        pltpu.sync_copy(pts_hbm.at[pl.ds(NFULL * 8192, TAILP * 4)],
                        pbuf.at[pl.ds(0, TAILP * 4)])
        pltpu.sync_copy(lin_hbm.at[pl.ds(NFULL * 2048, TAILP)],
                        rbuf.at[pl.ds(0, TAILP)])
        return emit_chunk(TAILP // 16, kc, append, pbuf, 0)

    def append_list(kc, keep, rel, x, y, z, w):
        pc2 = plsc.cumsum(keep.astype(jnp.int32))
        incap = keep & ((kc + pc2 - 1) < CAP)
        base = jnp.minimum(kc, CAP)
        plsc.store_compressed(ks_s.at[pl.ds(base, 16)], rel, mask=incap)
        plsc.store_compressed(ks_x.at[pl.ds(base, 16)], x, mask=incap)
        plsc.store_compressed(ks_y.at[pl.ds(base, 16)], y, mask=incap)
        plsc.store_compressed(ks_z.at[pl.ds(base, 16)], z, mask=incap)
        plsc.store_compressed(ks_w.at[pl.ds(base, 16)], w, mask=incap)
        return kc + jnp.sum(keep.astype(jnp.int32))

    kept = emit_pass(jnp.int32(0), append_list)

    kcl = jnp.minimum(kept, CAP)
    nwin = (nout + 255) // 256
    for q in range(4):
        cnst[pl.ds(q * 16, 16)] = zeros16

    def wh(i, _):
        m = (i * 16 + lane) < kcl
        w = lax.shift_right_logical(ks_s[pl.ds(i * 16, 16)], 13)
        plsc.addupdate_scatter(cnst, [jnp.where(m, w, 0)], ones, mask=m)
        return 0

    nkv = (kcl + 15) // 16
    lax.fori_loop(0, nkv, wh, 0)
    carry = jnp.int32(0)
    for q in range(4):
        cv = cnst[pl.ds(q * 16, 16)]
        excl = plsc.cumsum(cv) - cv + carry
        cnst[pl.ds(64 + q * 16, 16)] = excl
        cnst[pl.ds(128 + q * 16, 16)] = excl
        carry = carry + jnp.sum(cv)

    def wscat(i, _):
        iv = i * 16 + lane
        m = iv < kcl
        w = jnp.where(m, lax.shift_right_logical(ks_s[pl.ds(i * 16, 16)], 13),
                      0)
        prior, _u = plsc.scan_count(w, mask=m)
        base = plsc.load_gather(cnst, [64 + w], mask=m)
        plsc.addupdate_scatter(cnst, [64 + w], ones, mask=m)
        dst = jnp.minimum(base + prior - 1, CAP)
        plsc.store_scatter(ks_i, [jnp.where(m, dst, CAP)], iv, mask=m)
        return 0

    lax.fori_loop(0, nkv, wscat, 0)

    def p4(w, _):
        def wz(i, _):
            wstage[pl.ds(i * 16, 16)] = zf16
            return 0

        lax.fori_loop(0, 2048, wz, 0)
        sv = plsc.load_gather(cnst, [jnp.full((16,), 128, jnp.int32) + w])
        ev = plsc.load_gather(cnst, [jnp.full((16,), 64, jnp.int32) + w])
        s0 = jnp.max(sv)
        e0 = jnp.max(ev)

        def place(i, _):
            p = s0 + i * 16 + lane
            m = p < e0
            ki = plsc.load_gather(ks_i, [jnp.minimum(p, CAP)], mask=m)
            sl = plsc.load_gather(ks_s, [ki], mask=m)
            off = (sl - w * 8192) * 4
            off = jnp.where(m, off, 0)
            plsc.store_scatter(wstage, [off],
                               plsc.load_gather(ks_x, [ki], mask=m), mask=m)
            plsc.store_scatter(wstage, [off + 1],
                               plsc.load_gather(ks_y, [ki], mask=m), mask=m)
            plsc.store_scatter(wstage, [off + 2],
                               plsc.load_gather(ks_z, [ki], mask=m), mask=m)
            plsc.store_scatter(wstage, [off + 3],
                               plsc.load_gather(ks_w, [ki], mask=m), mask=m)
            return 0

        lax.fori_loop(0, (e0 - s0 + 15) // 16, place, 0)
        rows = jnp.minimum(nout - w * 256, 256)
        dst = pl.multiple_of((rb + w * 256) * 128, 8)

        @pl.when(rows == 256)
        def _():
            pltpu.sync_copy(wstage, vox_out.at[pl.ds(dst, 32768)])

        @pl.when(rows < 256)
        def _():
            def f16(q, _):
                pltpu.sync_copy(
                    wstage.at[pl.ds(pl.multiple_of(q * 2048, 8), 2048)],
                    vox_out.at[pl.ds(pl.multiple_of(dst + q * 2048, 8),
                                     2048)])
                return 0

            lax.fori_loop(0, rows // 16, f16, 0)
            r0 = rows // 16 * 16

            def f1(q, _):
                pltpu.sync_copy(
                    wstage.at[pl.ds(pl.multiple_of((r0 + q) * 128, 8), 128)],
                    vox_out.at[pl.ds(pl.multiple_of(dst + (r0 + q) * 128, 8),
                                     128)])
                return 0

            lax.fori_loop(0, rows - r0, f1, 0)

        return 0

    lax.fori_loop(0, nwin, p4, 0)

    def p5(i, cw):
        c0 = i * 16 + lane
        h = hcell[pl.ds(i * 16, 16)]
        cnt = h & CNTMASK
        lr = lax.shift_right_logical(h, 18)
        ok = (cnt > 0) & (lr < nout)
        g = lo + c0
        yv = g // GX
        xv = g - yv * GX
        off = jnp.where(ok, (lr - cw * 256) * 8, 2168)
        plsc.store_scatter(cnst, [off], zeros16, mask=ok)
        plsc.store_scatter(cnst, [off + 1], yv, mask=ok)
        plsc.store_scatter(cnst, [off + 2], xv, mask=ok)
        plsc.store_scatter(cnst, [off + 3], jnp.minimum(cnt, MAXP), mask=ok)
        hi = jnp.max(jnp.where(ok, lr, 0))
        crossed = hi >= (cw + 1) * 256

        @pl.when(crossed)
        def _():
            pltpu.sync_copy(
                cnst.at[pl.ds(0, 2048)],
                cn_hbm.at[pl.ds(pl.multiple_of((rb + cw * 256) * 8, 8),
                                2048)])
            for q in range(8):
                cnst[pl.ds(q * 16, 16)] = cnst[pl.ds(2048 + q * 16, 16)]

        return jnp.where(crossed, cw + 1, cw)

    cw = lax.fori_loop(0, NCV, p5, jnp.int32(0))
    rem = jnp.maximum(nout - cw * 256, 0)

    def fr16(q, _):
        pltpu.sync_copy(
            cnst.at[pl.ds(pl.multiple_of(q * 128, 8), 128)],
            cn_hbm.at[pl.ds(pl.multiple_of((rb + cw * 256 + q * 16) * 8, 8),
                            128)])
        return 0

    lax.fori_loop(0, rem // 16, fr16, 0)
    rr0 = rem // 16 * 16

    def fr1(q, _):
        pltpu.sync_copy(
            cnst.at[pl.ds(pl.multiple_of((rr0 + q) * 8, 8), 8)],
            cn_hbm.at[pl.ds(pl.multiple_of((rb + cw * 256 + rr0 + q) * 8, 8),
                            8)])
        return 0

    lax.fori_loop(0, rem - rr0, fr1, 0)

    @pl.when(kept > CAP)
    def _():
        def clr(i, _):
            h = hcell[pl.ds(i * 16, 16)]
            hcell[pl.ds(i * 16, 16)] = h & ~CNTMASK
            return 0

        lax.fori_loop(0, NCV, clr, 0)

        def append_ovf(kc, keep, rel, x, y, z, w):
            pc2 = plsc.cumsum(keep.astype(jnp.int32))
            ovf = keep & ((kc + pc2 - 1) >= CAP)

            @pl.when(jnp.sum(ovf.astype(jnp.int32)) > 0)
            def _():
                base = (rb * 128) + rel * 4
                for comp, val in ((0, x), (1, y), (2, z), (3, w)):
                    ovi[...] = jnp.where(ovf, base + comp, VOXDUMPW)
                    ovv[...] = val
                    pltpu.async_copy(ovv, vox_out.at[ovi], sem).wait()

            return kc + jnp.sum(keep.astype(jnp.int32))

        emit_pass(jnp.int32(0), append_ovf)

    plsc.subcore_barrier()

    for c in range(30):
        @pl.when(t == c % NT)
        def _(c=c):
            nw = 2048 if c < 29 else 608
            w0 = c * 2048
            row0 = w0 // 3
            pltpu.sync_copy(cn_hbm.at[pl.ds(row0 * 8, 5504)],
                            rbuf.at[pl.ds(0, 5504)])

            def rp(j, _):
                wd = w0 + j * 16 + lane
                r = wd // 3
                src = (r - row0) * 8 + (wd - r * 3)
                v = plsc.load_gather(rbuf, [src])
                ostage[pl.ds(j * 16, 16)] = jnp.where(r < vn, v, -1)
                return 0

            lax.fori_loop(0, nw // 16, rp, 0)
            pltpu.sync_copy(ostage.at[pl.ds(0, nw)],
                            coor_out.at[pl.ds(w0, nw)])

    for c in range(20):
        @pl.when(t == c % NT)
        def _(c=c):
            nw = 1024 if c < 19 else 544
            w0 = c * 1024
            pltpu.sync_copy(cn_hbm.at[pl.ds(w0 * 8, 8192)], rbuf)

            def rp(j, _):
                wd = w0 + j * 16 + lane
                src = (wd - w0) * 8 + 3
                v = plsc.load_gather(rbuf, [src])
                ostage[pl.ds(j * 16, 16)] = jnp.where(wd < vn, v, 0)
                return 0

            lax.fori_loop(0, nw // 16, rp, 0)
            pltpu.sync_copy(ostage.at[pl.ds(0, nw)],
                            npv_out.at[pl.ds(w0, nw)])


def kernel(points):
    pts_flat = points.reshape(-1)
    vox, coor, npv, vnum = _vox_kernel(pts_flat)
    voxels = vox[: MAXV * MAXP * C].reshape(MAXV, MAXP, C)
    coors = coor[: MAXV * 3].reshape(MAXV, 3)
    return voxels, coors, npv[:MAXV], vnum[0]


# speedup vs baseline: 1.2558x; 1.2558x over previous
import dataclasses
import functools

import jax
import jax.numpy as jnp
from jax import lax
from jax.experimental import pallas as pl
from jax.experimental.pallas import tpu as pltpu
from jax.experimental.pallas import tpu_sc as plsc

GX, GY = 432, 496
NCELL = GX * GY
MAXV, MAXP, C = 20000, 32, 4
N = 200000
NT = 16
CPT = NCELL // NT
NCV = CPT // 16
CAP = 8192

VOX_WORDS = 2560512
VOXDUMPW = 2560480
COOR_WORDS = 60160
NPV_WORDS = 20096
CN_ROWS = 20688

VS0, VS1, VS2 = 0.16, 0.16, 4.0
PR0, PR1, PR2 = 0.0, -39.68, -3.0
CNTMASK = 0x3FFFF

_MESH = plsc.VectorSubcoreMesh(core_axis_name="c", subcore_axis_name="s",
                               num_cores=1)
_CP = pltpu.CompilerParams()
if "needs_layout_passes" in pltpu.CompilerParams.__dataclass_fields__:
    _CP = dataclasses.replace(_CP, needs_layout_passes=False)

NFULL, TAILP = 97, 1344


def _floor_div(q):
    ti = q.astype(jnp.int32)
    return ti - (ti.astype(jnp.float32) > q).astype(jnp.int32)


@functools.partial(
    pl.kernel,
    out_type=[
        jax.ShapeDtypeStruct((VOX_WORDS,), jnp.float32),
        jax.ShapeDtypeStruct((COOR_WORDS,), jnp.int32),
        jax.ShapeDtypeStruct((NPV_WORDS,), jnp.int32),
        jax.ShapeDtypeStruct((16,), jnp.int32),
    ],
    mesh=_MESH,
    compiler_params=_CP,
    scratch_types=[
        pltpu.HBM((CN_ROWS * 8,), jnp.int32),
        pltpu.HBM((256,), jnp.int32),
        pltpu.HBM((200704,), jnp.int32),
        pltpu.VMEM((8192,), jnp.float32),
        pltpu.VMEM((8192,), jnp.float32),
        pltpu.VMEM((CPT,), jnp.int32),
        pltpu.VMEM((CAP + 16,), jnp.int32),
        pltpu.VMEM((CAP + 16,), jnp.float32),
        pltpu.VMEM((CAP + 16,), jnp.float32),
        pltpu.VMEM((CAP + 16,), jnp.float32),
        pltpu.VMEM((CAP + 16,), jnp.float32),
        pltpu.VMEM((CAP + 16,), jnp.int32),
        pltpu.VMEM((32768,), jnp.float32),
        pltpu.VMEM((2176,), jnp.int32),
        pltpu.VMEM((2048,), jnp.int32),
        pltpu.VMEM((8192,), jnp.int32),
        pltpu.VMEM((2048,), jnp.float32),
        pltpu.VMEM((16,), jnp.int32),
        pltpu.VMEM((16,), jnp.int32),
        pltpu.VMEM((16,), jnp.float32),
        pltpu.SemaphoreType.DMA,
        pltpu.SemaphoreType.DMA,
    ],
)
def _vox_kernel(pts_hbm, vox_out, coor_out, npv_out, vnum_out,
                cn_hbm, occ_hbm, lin_hbm,
                pbuf, pbuf2, hcell, ks_s, ks_x, ks_y, ks_z, ks_w, ks_i,
                wstage, cnst, ostage, rbuf, zf, b16, ovi, ovv, sem, fsem):
    t = lax.axis_index("s")
    lane = lax.iota(jnp.int32, 16)
    ones = jnp.ones((16,), jnp.int32)
    zeros16 = jnp.zeros((16,), jnp.int32)
    zf16 = jnp.zeros((16,), jnp.float32)
    lo = t * CPT

    def lin_of(j):
        idx = (j * 16 + lane) * 4
        x = plsc.load_gather(pbuf, [idx])
        y = plsc.load_gather(pbuf, [idx + 1])
        z = plsc.load_gather(pbuf, [idx + 2])
        cx = _floor_div((x - PR0) / jnp.float32(VS0))
        cy = _floor_div((y - PR1) / jnp.float32(VS1))
        cz = _floor_div((z - PR2) / jnp.float32(VS2))
        valid = ((cx >= 0) & (cx < GX) & (cy >= 0) & (cy < GY) & (cz == 0))
        return jnp.where(valid, cy * GX + cx, NCELL), x, y, z, idx

    def zb(i, _):
        zf[pl.ds(i * 16, 16)] = zf16
        return 0

    lax.fori_loop(0, 128, zb, 0)
    vz = pl.multiple_of(t * 160032, 8)
    for k in range(78):
        pltpu.async_copy(zf, vox_out.at[pl.ds(vz + k * 2048, 2048)], fsem)
    pltpu.async_copy(zf.at[pl.ds(0, 288)],
                     vox_out.at[pl.ds(vz + 78 * 2048, 288)], fsem)

    def hz(i, _):
        hcell[pl.ds(i * 16, 16)] = zeros16
        return 0

    lax.fori_loop(0, NCV, hz, 0)

    p0s = pl.multiple_of(t * 12512, 8)

    def p0_chunk(c0, nvec):
        def body(j, _):
            lin, _x, _y, _z, _i = lin_of(j)
            ostage[pl.ds(j * 16, 16)] = lin
            return 0

        lax.fori_loop(0, nvec, body, 0)

    def p0(c0, _):
        pltpu.sync_copy(pts_hbm.at[pl.ds(pl.multiple_of((p0s + c0 * 2048) * 4,
                                                        8), 8192)], pbuf)
        p0_chunk(c0, 128)
        pltpu.sync_copy(ostage,
                        lin_hbm.at[pl.ds(pl.multiple_of(p0s + c0 * 2048, 8),
                                         2048)])
        return 0

    lax.fori_loop(0, 6, p0, 0)

    @pl.when(t < 15)
    def _():
        pltpu.sync_copy(pts_hbm.at[pl.ds(pl.multiple_of((p0s + 12288) * 4, 8),
                                         896)], pbuf.at[pl.ds(0, 896)])
        p0_chunk(6, 14)
        pltpu.sync_copy(ostage.at[pl.ds(0, 224)],
                        lin_hbm.at[pl.ds(pl.multiple_of(p0s + 12288, 8), 224)])

    @pl.when(t == 15)
    def _():
        pltpu.sync_copy(pts_hbm.at[pl.ds(pl.multiple_of((p0s + 12288) * 4, 8),
                                         128)], pbuf.at[pl.ds(0, 128)])
        p0_chunk(6, 2)
        pltpu.sync_copy(ostage.at[pl.ds(0, 32)],
                        lin_hbm.at[pl.ds(pl.multiple_of(p0s + 12288, 8), 32)])

    plsc.subcore_barrier()

    def p1_chunk(nvec, boff):
        def body(j, _):
            lin = rbuf[pl.ds(boff + j * 16, 16)]
            inr = (lin >= lo) & (lin < lo + CPT)
            cell = jnp.where(inr, lin - lo, 0)
            plsc.addupdate_scatter(hcell, [cell], ones, mask=inr)
            return 0

        lax.fori_loop(0, nvec, body, 0)

    def lin_cp(c0, half):
        return pltpu.async_copy(
            lin_hbm.at[pl.ds(pl.multiple_of(c0 * 4096, 8), 4096)],
            rbuf.at[pl.ds(half * 4096, 4096)], sem)

    lin_cp(0, 0)
    lin_cp(1, 1)

    def p1(k, _):
        for h in range(2):
            c = 2 * k + h
            pltpu.make_async_copy(
                lin_hbm.at[pl.ds(pl.multiple_of(c * 4096, 8), 4096)],
                rbuf.at[pl.ds(h * 4096, 4096)], sem).wait()
            p1_chunk(256, h * 4096)

            @pl.when(c + 2 < 48)
            def _(c=c, h=h):
                lin_cp(c + 2, h)

        return 0

    lax.fori_loop(0, 24, p1, 0)
    pltpu.sync_copy(lin_hbm.at[pl.ds(48 * 4096, 3392)],
                    rbuf.at[pl.ds(0, 3392)])
    p1_chunk(212, 0)

    def p2(i, carry):
        h = hcell[pl.ds(i * 16, 16)]
        occ = (h > 0).astype(jnp.int32)
        excl = plsc.cumsum(occ) - occ + carry
        hcell[pl.ds(i * 16, 16)] = excl << 18
        return carry + jnp.sum(occ)

    occ_t = lax.fori_loop(0, NCV, p2, jnp.int32(0))
    b16[...] = jnp.full((16,), occ_t, jnp.int32)
    pltpu.sync_copy(b16, occ_hbm.at[pl.ds(pl.multiple_of(t * 16, 8), 16)])

    for k in range(78):
        pltpu.make_async_copy(zf, vox_out.at[pl.ds(vz + k * 2048, 2048)],
                              fsem).wait()
    pltpu.make_async_copy(zf.at[pl.ds(0, 288)],
                          vox_out.at[pl.ds(vz + 78 * 2048, 288)], fsem).wait()

    plsc.subcore_barrier()
    pltpu.sync_copy(occ_hbm, cnst.at[pl.ds(0, 256)])
    occv = plsc.load_gather(cnst, [lane * 16])
    rb = jnp.sum(jnp.where(lane < t, occv, 0))
    total_occ = jnp.sum(occv)
    vn = jnp.minimum(total_occ, MAXV)
    nout = jnp.clip(jnp.minimum(occ_t, MAXV - rb), 0, MAXV)

    @pl.when(t == 0)
    def _():
        b16[...] = jnp.full((16,), vn, jnp.int32)
        pltpu.sync_copy(b16, vnum_out)

    def emit_chunk(nvec, kc0, append, pb, loff):
        def body(j, kc):
            idx = (j * 16 + lane) * 4
            lin = rbuf[pl.ds(loff + j * 16, 16)]
            x = plsc.load_gather(pb, [idx])
            y = plsc.load_gather(pb, [idx + 1])
            z = plsc.load_gather(pb, [idx + 2])
            w = plsc.load_gather(pb, [idx + 3])
            inr = (lin >= lo) & (lin < lo + CPT)
            cell = jnp.where(inr, lin - lo, 0)
            h = plsc.load_gather(hcell, [cell], mask=inr)
            prior, _u = plsc.scan_count(cell, mask=inr)
            pos = (h & CNTMASK) + prior - 1
            lr = lax.shift_right_logical(h, 18)
            plsc.addupdate_scatter(hcell, [cell], ones, mask=inr)
            keep = inr & (pos < MAXP) & (lr < nout)
            rel = lr * MAXP + pos
            return append(kc, keep, rel, x, y, z, w)

        return lax.fori_loop(0, nvec, body, kc0)

    def emit_pass(kc0, append):
        pbufs = (pbuf, pbuf2)

        def pt_cp(c0, h):
            pltpu.async_copy(
                pts_hbm.at[pl.ds(pl.multiple_of(c0 * 8192, 8), 8192)],
                pbufs[h], sem)
            pltpu.async_copy(
                lin_hbm.at[pl.ds(pl.multiple_of(c0 * 2048, 8), 2048)],
                rbuf.at[pl.ds(h * 2048, 2048)], sem)

        def pt_wait(c0, h):
            pltpu.make_async_copy(
                pts_hbm.at[pl.ds(pl.multiple_of(c0 * 8192, 8), 8192)],
                pbufs[h], sem).wait()
            pltpu.make_async_copy(
                lin_hbm.at[pl.ds(pl.multiple_of(c0 * 2048, 8), 2048)],
                rbuf.at[pl.ds(h * 2048, 2048)], sem).wait()

        pt_cp(0, 0)
        pt_cp(1, 1)

        def pc(k, kc):
            for h in range(2):
                c = 2 * k + h
                pt_wait(c, h)
                kc = emit_chunk(128, kc, append, pbufs[h], h * 2048)

                @pl.when(c + 2 < 96)
                def _(c=c, h=h):
                    pt_cp(c + 2, h)

            return kc

        kc = lax.fori_loop(0, 48, pc, kc0)
        pltpu.sync_copy(pts_hbm.at[pl.ds(96 * 8192, 8192)], pbuf)
        pltpu.sync_copy(lin_hbm.at[pl.ds(96 * 2048, 2048)],
                        rbuf.at[pl.ds(0, 2048)])
        kc = emit_chunk(128, kc, append, pbuf, 0)
        pltpu.sync_copy(pts_hbm.at[pl.ds(NFULL * 8192, TAILP * 4)],
                        pbuf.at[pl.ds(0, TAILP * 4)])
        pltpu.sync_copy(lin_hbm.at[pl.ds(NFULL * 2048, TAILP)],
                        rbuf.at[pl.ds(0, TAILP)])
        return emit_chunk(TAILP // 16, kc, append, pbuf, 0)

    def append_list(kc, keep, rel, x, y, z, w):
        pc2 = plsc.cumsum(keep.astype(jnp.int32))
        incap = keep & ((kc + pc2 - 1) < CAP)
        base = jnp.minimum(kc, CAP)
        plsc.store_compressed(ks_s.at[pl.ds(base, 16)], rel, mask=incap)
        plsc.store_compressed(ks_x.at[pl.ds(base, 16)], x, mask=incap)
        plsc.store_compressed(ks_y.at[pl.ds(base, 16)], y, mask=incap)
        plsc.store_compressed(ks_z.at[pl.ds(base, 16)], z, mask=incap)
        plsc.store_compressed(ks_w.at[pl.ds(base, 16)], w, mask=incap)
        return kc + jnp.sum(keep.astype(jnp.int32))

    kept = emit_pass(jnp.int32(0), append_list)

    kcl = jnp.minimum(kept, CAP)
    nwin = (nout + 255) // 256
    for q in range(4):
        cnst[pl.ds(q * 16, 16)] = zeros16

    def wh(i, _):
        m = (i * 16 + lane) < kcl
        w = lax.shift_right_logical(ks_s[pl.ds(i * 16, 16)], 13)
        plsc.addupdate_scatter(cnst, [jnp.where(m, w, 0)], ones, mask=m)
        return 0

    nkv = (kcl + 15) // 16
    lax.fori_loop(0, nkv, wh, 0)
    carry = jnp.int32(0)
    for q in range(4):
        cv = cnst[pl.ds(q * 16, 16)]
        excl = plsc.cumsum(cv) - cv + carry
        cnst[pl.ds(64 + q * 16, 16)] = excl
        cnst[pl.ds(128 + q * 16, 16)] = excl
        carry = carry + jnp.sum(cv)

    def wscat(i, _):
        iv = i * 16 + lane
        m = iv < kcl
        w = jnp.where(m, lax.shift_right_logical(ks_s[pl.ds(i * 16, 16)], 13),
                      0)
        prior, _u = plsc.scan_count(w, mask=m)
        base = plsc.load_gather(cnst, [64 + w], mask=m)
        plsc.addupdate_scatter(cnst, [64 + w], ones, mask=m)
        dst = jnp.minimum(base + prior - 1, CAP)
        plsc.store_scatter(ks_i, [jnp.where(m, dst, CAP)], iv, mask=m)
        return 0

    lax.fori_loop(0, nkv, wscat, 0)

    def p4(w, _):
        def wz(i, _):
            wstage[pl.ds(i * 16, 16)] = zf16
            return 0

        lax.fori_loop(0, 2048, wz, 0)
        sv = plsc.load_gather(cnst, [jnp.full((16,), 128, jnp.int32) + w])
        ev = plsc.load_gather(cnst, [jnp.full((16,), 64, jnp.int32) + w])
        s0 = jnp.max(sv)
        e0 = jnp.max(ev)

        def place(i, _):
            p = s0 + i * 16 + lane
            m = p < e0
            ki = plsc.load_gather(ks_i, [jnp.minimum(p, CAP)], mask=m)
            sl = plsc.load_gather(ks_s, [ki], mask=m)
            off = (sl - w * 8192) * 4
            off = jnp.where(m, off, 0)
            plsc.store_scatter(wstage, [off],
                               plsc.load_gather(ks_x, [ki], mask=m), mask=m)
            plsc.store_scatter(wstage, [off + 1],
                               plsc.load_gather(ks_y, [ki], mask=m), mask=m)
            plsc.store_scatter(wstage, [off + 2],
                               plsc.load_gather(ks_z, [ki], mask=m), mask=m)
            plsc.store_scatter(wstage, [off + 3],
                               plsc.load_gather(ks_w, [ki], mask=m), mask=m)
            return 0

        lax.fori_loop(0, (e0 - s0 + 15) // 16, place, 0)
        rows = jnp.minimum(nout - w * 256, 256)
        dst = pl.multiple_of((rb + w * 256) * 128, 8)

        @pl.when(rows == 256)
        def _():
            pltpu.sync_copy(wstage, vox_out.at[pl.ds(dst, 32768)])

        @pl.when(rows < 256)
        def _():
            def f16(q, _):
                pltpu.sync_copy(
                    wstage.at[pl.ds(pl.multiple_of(q * 2048, 8), 2048)],
                    vox_out.at[pl.ds(pl.multiple_of(dst + q * 2048, 8),
                                     2048)])
                return 0

            lax.fori_loop(0, rows // 16, f16, 0)
            r0 = rows // 16 * 16

            def f1(q, _):
                pltpu.sync_copy(
                    wstage.at[pl.ds(pl.multiple_of((r0 + q) * 128, 8), 128)],
                    vox_out.at[pl.ds(pl.multiple_of(dst + (r0 + q) * 128, 8),
                                     128)])
                return 0

            lax.fori_loop(0, rows - r0, f1, 0)

        return 0

    lax.fori_loop(0, nwin, p4, 0)

    def p5(i, cw):
        c0 = i * 16 + lane
        h = hcell[pl.ds(i * 16, 16)]
        cnt = h & CNTMASK
        lr = lax.shift_right_logical(h, 18)
        ok = (cnt > 0) & (lr < nout)
        g = lo + c0
        yv = g // GX
        xv = g - yv * GX
        off = jnp.where(ok, (lr - cw * 256) * 8, 2168)
        plsc.store_scatter(cnst, [off], zeros16, mask=ok)
        plsc.store_scatter(cnst, [off + 1], yv, mask=ok)
        plsc.store_scatter(cnst, [off + 2], xv, mask=ok)
        plsc.store_scatter(cnst, [off + 3], jnp.minimum(cnt, MAXP), mask=ok)
        hi = jnp.max(jnp.where(ok, lr, 0))
        crossed = hi >= (cw + 1) * 256

        @pl.when(crossed)
        def _():
            pltpu.sync_copy(
                cnst.at[pl.ds(0, 2048)],
                cn_hbm.at[pl.ds(pl.multiple_of((rb + cw * 256) * 8, 8),
                                2048)])
            for q in range(8):
                cnst[pl.ds(q * 16, 16)] = cnst[pl.ds(2048 + q * 16, 16)]

        return jnp.where(crossed, cw + 1, cw)

    cw = lax.fori_loop(0, NCV, p5, jnp.int32(0))
    rem = jnp.maximum(nout - cw * 256, 0)

    def fr16(q, _):
        pltpu.sync_copy(
            cnst.at[pl.ds(pl.multiple_of(q * 128, 8), 128)],
            cn_hbm.at[pl.ds(pl.multiple_of((rb + cw * 256 + q * 16) * 8, 8),
                            128)])
        return 0

    lax.fori_loop(0, rem // 16, fr16, 0)
    rr0 = rem // 16 * 16

    def fr1(q, _):
        pltpu.sync_copy(
            cnst.at[pl.ds(pl.multiple_of((rr0 + q) * 8, 8), 8)],
            cn_hbm.at[pl.ds(pl.multiple_of((rb + cw * 256 + rr0 + q) * 8, 8),
                            8)])
        return 0

    lax.fori_loop(0, rem - rr0, fr1, 0)

    @pl.when(kept > CAP)
    def _():
        def clr(i, _):
            h = hcell[pl.ds(i * 16, 16)]
            hcell[pl.ds(i * 16, 16)] = h & ~CNTMASK
            return 0

        lax.fori_loop(0, NCV, clr, 0)

        def append_ovf(kc, keep, rel, x, y, z, w):
            pc2 = plsc.cumsum(keep.astype(jnp.int32))
            ovf = keep & ((kc + pc2 - 1) >= CAP)

            @pl.when(jnp.sum(ovf.astype(jnp.int32)) > 0)
            def _():
                base = (rb * 128) + rel * 4
                for comp, val in ((0, x), (1, y), (2, z), (3, w)):
                    ovi[...] = jnp.where(ovf, base + comp, VOXDUMPW)
                    ovv[...] = val
                    pltpu.async_copy(ovv, vox_out.at[ovi], sem).wait()

            return kc + jnp.sum(keep.astype(jnp.int32))

        emit_pass(jnp.int32(0), append_ovf)

    plsc.subcore_barrier()

    for c in range(30):
        @pl.when(t == c % NT)
        def _(c=c):
            nw = 2048 if c < 29 else 608
            w0 = c * 2048
            row0 = w0 // 3
            pltpu.sync_copy(cn_hbm.at[pl.ds(row0 * 8, 5504)],
                            rbuf.at[pl.ds(0, 5504)])

            def rp(j, _):
                wd = w0 + j * 16 + lane
                r = wd // 3
                src = (r - row0) * 8 + (wd - r * 3)
                v = plsc.load_gather(rbuf, [src])
                ostage[pl.ds(j * 16, 16)] = jnp.where(r < vn, v, -1)
                return 0

            lax.fori_loop(0, nw // 16, rp, 0)
            pltpu.sync_copy(ostage.at[pl.ds(0, nw)],
                            coor_out.at[pl.ds(w0, nw)])

    for c in range(20):
        @pl.when(t == c % NT)
        def _(c=c):
            nw = 1024 if c < 19 else 544
            w0 = c * 1024
            pltpu.sync_copy(cn_hbm.at[pl.ds(w0 * 8, 8192)], rbuf)

            def rp(j, _):
                wd = w0 + j * 16 + lane
                src = (wd - w0) * 8 + 3
                v = plsc.load_gather(rbuf, [src])
                ostage[pl.ds(j * 16, 16)] = jnp.where(wd < vn, v, 0)
                return 0

            lax.fori_loop(0, nw // 16, rp, 0)
            pltpu.sync_copy(ostage.at[pl.ds(0, nw)],
                            npv_out.at[pl.ds(w0, nw)])


def kernel(points):
    pts_flat = points.reshape(-1)
    vox, coor, npv, vnum = _vox_kernel(pts_flat)
    voxels = vox[: MAXV * MAXP * C].reshape(MAXV, MAXP, C)
    coors = coor[: MAXV * 3].reshape(MAXV, 3)
    return voxels, coors, npv[:MAXV], vnum[0]


# speedup vs baseline: 1.4230x; 1.1331x over previous
import dataclasses
import functools

import jax
import jax.numpy as jnp
from jax import lax
from jax.experimental import pallas as pl
from jax.experimental.pallas import tpu as pltpu
from jax.experimental.pallas import tpu_sc as plsc

GX, GY = 432, 496
NCELL = GX * GY
MAXV, MAXP, C = 20000, 32, 4
N = 200000
NT = 16
CPT = NCELL // NT
NCV = CPT // 16
CAP = 8192

VOX_WORDS = 2560512
VOXDUMPW = 2560480
COOR_WORDS = 60160
NPV_WORDS = 20096
CN_ROWS = 20688

VS0, VS1, VS2 = 0.16, 0.16, 4.0
PR0, PR1, PR2 = 0.0, -39.68, -3.0
CNTMASK = 0x3FFFF

_MESH = plsc.VectorSubcoreMesh(core_axis_name="c", subcore_axis_name="s",
                               num_cores=1)
_CP = pltpu.CompilerParams()
if "needs_layout_passes" in pltpu.CompilerParams.__dataclass_fields__:
    _CP = dataclasses.replace(_CP, needs_layout_passes=False)

NFULL, TAILP = 97, 1344


def _floor_div(q):
    ti = q.astype(jnp.int32)
    return ti - (ti.astype(jnp.float32) > q).astype(jnp.int32)


@functools.partial(
    pl.kernel,
    out_type=[
        jax.ShapeDtypeStruct((VOX_WORDS,), jnp.float32),
        jax.ShapeDtypeStruct((COOR_WORDS,), jnp.int32),
        jax.ShapeDtypeStruct((NPV_WORDS,), jnp.int32),
        jax.ShapeDtypeStruct((16,), jnp.int32),
    ],
    mesh=_MESH,
    compiler_params=_CP,
    scratch_types=[
        pltpu.HBM((CN_ROWS * 8,), jnp.int32),
        pltpu.HBM((256,), jnp.int32),
        pltpu.HBM((200704,), jnp.int32),
        pltpu.VMEM((8192,), jnp.float32),
        pltpu.VMEM((8192,), jnp.float32),
        pltpu.VMEM((CPT,), jnp.int32),
        pltpu.VMEM((CAP + 16,), jnp.int32),
        pltpu.VMEM((CAP + 16,), jnp.float32),
        pltpu.VMEM((CAP + 16,), jnp.float32),
        pltpu.VMEM((CAP + 16,), jnp.float32),
        pltpu.VMEM((CAP + 16,), jnp.float32),
        pltpu.VMEM((CAP + 16,), jnp.int32),
        pltpu.VMEM((32768,), jnp.float32),
        pltpu.VMEM((2176,), jnp.int32),
        pltpu.VMEM((2048,), jnp.int32),
        pltpu.VMEM((8192,), jnp.int32),
        pltpu.VMEM((2048,), jnp.float32),
        pltpu.VMEM((16,), jnp.int32),
        pltpu.VMEM((16,), jnp.int32),
        pltpu.VMEM((16,), jnp.float32),
        pltpu.SemaphoreType.DMA,
        pltpu.SemaphoreType.DMA,
    ],
)
def _vox_kernel(pts_hbm, vox_out, coor_out, npv_out, vnum_out,
                cn_hbm, occ_hbm, lin_hbm,
                pbuf, pbuf2, hcell, ks_s, ks_x, ks_y, ks_z, ks_w, ks_i,
                wstage, cnst, ostage, rbuf, zf, b16, ovi, ovv, sem, fsem):
    t = lax.axis_index("s")
    lane = lax.iota(jnp.int32, 16)
    ones = jnp.ones((16,), jnp.int32)
    zeros16 = jnp.zeros((16,), jnp.int32)
    zf16 = jnp.zeros((16,), jnp.float32)
    lo = t * CPT

    def lin_of(j):
        idx = (j * 16 + lane) * 4
        x = plsc.load_gather(pbuf, [idx])
        y = plsc.load_gather(pbuf, [idx + 1])
        z = plsc.load_gather(pbuf, [idx + 2])
        cx = _floor_div((x - PR0) / jnp.float32(VS0))
        cy = _floor_div((y - PR1) / jnp.float32(VS1))
        cz = _floor_div((z - PR2) / jnp.float32(VS2))
        valid = ((cx >= 0) & (cx < GX) & (cy >= 0) & (cy < GY) & (cz == 0))
        return jnp.where(valid, cy * GX + cx, NCELL), x, y, z, idx

    def zb(i, _):
        zf[pl.ds(i * 16, 16)] = zf16
        return 0

    lax.fori_loop(0, 128, zb, 0)
    vz = pl.multiple_of(t * 160032, 8)
    for k in range(78):
        pltpu.async_copy(zf, vox_out.at[pl.ds(vz + k * 2048, 2048)], fsem)
    pltpu.async_copy(zf.at[pl.ds(0, 288)],
                     vox_out.at[pl.ds(vz + 78 * 2048, 288)], fsem)

    def hz(i, _):
        hcell[pl.ds(i * 16, 16)] = zeros16
        return 0

    lax.fori_loop(0, NCV, hz, 0)

    p0s = pl.multiple_of(t * 12512, 8)

    def p0_chunk(c0, nvec):
        def body(j, _):
            lin, _x, _y, _z, _i = lin_of(j)
            ostage[pl.ds(j * 16, 16)] = lin
            return 0

        lax.fori_loop(0, nvec, body, 0)

    def p0(c0, _):
        pltpu.sync_copy(pts_hbm.at[pl.ds(pl.multiple_of((p0s + c0 * 2048) * 4,
                                                        8), 8192)], pbuf)
        p0_chunk(c0, 128)
        pltpu.sync_copy(ostage,
                        lin_hbm.at[pl.ds(pl.multiple_of(p0s + c0 * 2048, 8),
                                         2048)])
        return 0

    lax.fori_loop(0, 6, p0, 0)

    @pl.when(t < 15)
    def _():
        pltpu.sync_copy(pts_hbm.at[pl.ds(pl.multiple_of((p0s + 12288) * 4, 8),
                                         896)], pbuf.at[pl.ds(0, 896)])
        p0_chunk(6, 14)
        pltpu.sync_copy(ostage.at[pl.ds(0, 224)],
                        lin_hbm.at[pl.ds(pl.multiple_of(p0s + 12288, 8), 224)])

    @pl.when(t == 15)
    def _():
        pltpu.sync_copy(pts_hbm.at[pl.ds(pl.multiple_of((p0s + 12288) * 4, 8),
                                         128)], pbuf.at[pl.ds(0, 128)])
        p0_chunk(6, 2)
        pltpu.sync_copy(ostage.at[pl.ds(0, 32)],
                        lin_hbm.at[pl.ds(pl.multiple_of(p0s + 12288, 8), 32)])

    plsc.subcore_barrier()

    def p1_chunk(nvec, boff):
        def body(j, _):
            lin = rbuf[pl.ds(boff + j * 16, 16)]
            inr = (lin >= lo) & (lin < lo + CPT)
            cell = jnp.where(inr, lin - lo, 0)
            plsc.addupdate_scatter(hcell, [cell], ones, mask=inr)
            return 0

        lax.fori_loop(0, nvec, body, 0)

    def lin_cp(c0, half):
        return pltpu.async_copy(
            lin_hbm.at[pl.ds(pl.multiple_of(c0 * 4096, 8), 4096)],
            rbuf.at[pl.ds(half * 4096, 4096)], sem)

    lin_cp(0, 0)
    lin_cp(1, 1)

    def p1(k, _):
        for h in range(2):
            c = 2 * k + h
            pltpu.make_async_copy(
                lin_hbm.at[pl.ds(pl.multiple_of(c * 4096, 8), 4096)],
                rbuf.at[pl.ds(h * 4096, 4096)], sem).wait()
            p1_chunk(256, h * 4096)

            @pl.when(c + 2 < 48)
            def _(c=c, h=h):
                lin_cp(c + 2, h)

        return 0

    lax.fori_loop(0, 24, p1, 0)
    pltpu.sync_copy(lin_hbm.at[pl.ds(48 * 4096, 3392)],
                    rbuf.at[pl.ds(0, 3392)])
    p1_chunk(212, 0)

    def p2(i, carry):
        h = hcell[pl.ds(i * 16, 16)]
        occ = (h > 0).astype(jnp.int32)
        excl = plsc.cumsum(occ) - occ + carry
        hcell[pl.ds(i * 16, 16)] = excl << 18
        return carry + jnp.sum(occ)

    occ_t = lax.fori_loop(0, NCV, p2, jnp.int32(0))
    b16[...] = jnp.full((16,), occ_t, jnp.int32)
    pltpu.sync_copy(b16, occ_hbm.at[pl.ds(pl.multiple_of(t * 16, 8), 16)])

    for k in range(78):
        pltpu.make_async_copy(zf, vox_out.at[pl.ds(vz + k * 2048, 2048)],
                              fsem).wait()
    pltpu.make_async_copy(zf.at[pl.ds(0, 288)],
                          vox_out.at[pl.ds(vz + 78 * 2048, 288)], fsem).wait()

    plsc.subcore_barrier()
    pltpu.sync_copy(occ_hbm, cnst.at[pl.ds(0, 256)])
    occv = plsc.load_gather(cnst, [lane * 16])
    rb = jnp.sum(jnp.where(lane < t, occv, 0))
    total_occ = jnp.sum(occv)
    vn = jnp.minimum(total_occ, MAXV)
    nout = jnp.clip(jnp.minimum(occ_t, MAXV - rb), 0, MAXV)

    @pl.when(t == 0)
    def _():
        b16[...] = jnp.full((16,), vn, jnp.int32)
        pltpu.sync_copy(b16, vnum_out)

    def emit_chunk(nvec, kc0, append, pb, loff):
        def body(j, kc):
            idx = (j * 16 + lane) * 4
            lin = rbuf[pl.ds(loff + j * 16, 16)]
            x = plsc.load_gather(pb, [idx])
            y = plsc.load_gather(pb, [idx + 1])
            z = plsc.load_gather(pb, [idx + 2])
            w = plsc.load_gather(pb, [idx + 3])
            inr = (lin >= lo) & (lin < lo + CPT)
            cell = jnp.where(inr, lin - lo, 0)
            h = plsc.load_gather(hcell, [cell], mask=inr)
            prior, _u = plsc.scan_count(cell, mask=inr)
            pos = (h & CNTMASK) + prior - 1
            lr = lax.shift_right_logical(h, 18)
            plsc.addupdate_scatter(hcell, [cell], ones, mask=inr)
            keep = inr & (pos < MAXP) & (lr < nout)
            rel = lr * MAXP + pos
            return append(kc, keep, rel, x, y, z, w)

        return lax.fori_loop(0, nvec, body, kc0)

    def emit_pass(kc0, append):
        pbufs = (pbuf, pbuf2)

        def pt_cp(c0, h):
            pltpu.async_copy(
                pts_hbm.at[pl.ds(pl.multiple_of(c0 * 8192, 8), 8192)],
                pbufs[h], sem)
            pltpu.async_copy(
                lin_hbm.at[pl.ds(pl.multiple_of(c0 * 2048, 8), 2048)],
                rbuf.at[pl.ds(h * 2048, 2048)], sem)

        def pt_wait(c0, h):
            pltpu.make_async_copy(
                pts_hbm.at[pl.ds(pl.multiple_of(c0 * 8192, 8), 8192)],
                pbufs[h], sem).wait()
            pltpu.make_async_copy(
                lin_hbm.at[pl.ds(pl.multiple_of(c0 * 2048, 8), 2048)],
                rbuf.at[pl.ds(h * 2048, 2048)], sem).wait()

        pt_cp(0, 0)
        pt_cp(1, 1)

        def pc(k, kc):
            for h in range(2):
                c = 2 * k + h
                pt_wait(c, h)
                kc = emit_chunk(128, kc, append, pbufs[h], h * 2048)

                @pl.when(c + 2 < 96)
                def _(c=c, h=h):
                    pt_cp(c + 2, h)

            return kc

        kc = lax.fori_loop(0, 48, pc, kc0)
        pltpu.sync_copy(pts_hbm.at[pl.ds(96 * 8192, 8192)], pbuf)
        pltpu.sync_copy(lin_hbm.at[pl.ds(96 * 2048, 2048)],
                        rbuf.at[pl.ds(0, 2048)])
        kc = emit_chunk(128, kc, append, pbuf, 0)
        pltpu.sync_copy(pts_hbm.at[pl.ds(NFULL * 8192, TAILP * 4)],
                        pbuf.at[pl.ds(0, TAILP * 4)])
        pltpu.sync_copy(lin_hbm.at[pl.ds(NFULL * 2048, TAILP)],
                        rbuf.at[pl.ds(0, TAILP)])
        return emit_chunk(TAILP // 16, kc, append, pbuf, 0)

    def append_list(kc, keep, rel, x, y, z, w):
        pc2 = plsc.cumsum(keep.astype(jnp.int32))
        incap = keep & ((kc + pc2 - 1) < CAP)
        base = jnp.minimum(kc, CAP)
        plsc.store_compressed(ks_s.at[pl.ds(base, 16)], rel, mask=incap)
        plsc.store_compressed(ks_x.at[pl.ds(base, 16)], x, mask=incap)
        plsc.store_compressed(ks_y.at[pl.ds(base, 16)], y, mask=incap)
        plsc.store_compressed(ks_z.at[pl.ds(base, 16)], z, mask=incap)
        plsc.store_compressed(ks_w.at[pl.ds(base, 16)], w, mask=incap)
        return kc + jnp.sum(keep.astype(jnp.int32))

    kept = emit_pass(jnp.int32(0), append_list)

    kcl = jnp.minimum(kept, CAP)
    nwin = (nout + 255) // 256
    for q in range(4):
        cnst[pl.ds(q * 16, 16)] = zeros16

    def wh(i, _):
        m = (i * 16 + lane) < kcl
        w = lax.shift_right_logical(ks_s[pl.ds(i * 16, 16)], 13)
        plsc.addupdate_scatter(cnst, [jnp.where(m, w, 0)], ones, mask=m)
        return 0

    nkv = (kcl + 15) // 16
    lax.fori_loop(0, nkv, wh, 0)
    carry = jnp.int32(0)
    for q in range(4):
        cv = cnst[pl.ds(q * 16, 16)]
        excl = plsc.cumsum(cv) - cv + carry
        cnst[pl.ds(64 + q * 16, 16)] = excl
        cnst[pl.ds(128 + q * 16, 16)] = excl
        carry = carry + jnp.sum(cv)

    def wscat(i, _):
        iv = i * 16 + lane
        m = iv < kcl
        w = jnp.where(m, lax.shift_right_logical(ks_s[pl.ds(i * 16, 16)], 13),
                      0)
        prior, _u = plsc.scan_count(w, mask=m)
        base = plsc.load_gather(cnst, [64 + w], mask=m)
        plsc.addupdate_scatter(cnst, [64 + w], ones, mask=m)
        dst = jnp.minimum(base + prior - 1, CAP)
        plsc.store_scatter(ks_i, [jnp.where(m, dst, CAP)], iv, mask=m)
        return 0

    lax.fori_loop(0, nkv, wscat, 0)

    def wz(i, _):
        wstage[pl.ds(i * 16, 16)] = zf16
        return 0

    lax.fori_loop(0, 2048, wz, 0)

    def p4(w, _):
        sv = plsc.load_gather(cnst, [jnp.full((16,), 128, jnp.int32) + w])
        ev = plsc.load_gather(cnst, [jnp.full((16,), 64, jnp.int32) + w])
        s0 = jnp.max(sv)
        e0 = jnp.max(ev)

        def place(i, _):
            p = s0 + i * 16 + lane
            m = p < e0
            ki = plsc.load_gather(ks_i, [jnp.minimum(p, CAP)], mask=m)
            sl = plsc.load_gather(ks_s, [ki], mask=m)
            off = (sl - w * 8192) * 4
            off = jnp.where(m, off, 0)
            plsc.store_scatter(wstage, [off],
                               plsc.load_gather(ks_x, [ki], mask=m), mask=m)
            plsc.store_scatter(wstage, [off + 1],
                               plsc.load_gather(ks_y, [ki], mask=m), mask=m)
            plsc.store_scatter(wstage, [off + 2],
                               plsc.load_gather(ks_z, [ki], mask=m), mask=m)
            plsc.store_scatter(wstage, [off + 3],
                               plsc.load_gather(ks_w, [ki], mask=m), mask=m)
            return 0

        lax.fori_loop(0, (e0 - s0 + 15) // 16, place, 0)
        rows = jnp.minimum(nout - w * 256, 256)
        dst = pl.multiple_of((rb + w * 256) * 128, 8)

        @pl.when(rows == 256)
        def _():
            pltpu.sync_copy(wstage, vox_out.at[pl.ds(dst, 32768)])

        @pl.when(rows < 256)
        def _():
            def f16(q, _):
                pltpu.sync_copy(
                    wstage.at[pl.ds(pl.multiple_of(q * 2048, 8), 2048)],
                    vox_out.at[pl.ds(pl.multiple_of(dst + q * 2048, 8),
                                     2048)])
                return 0

            lax.fori_loop(0, rows // 16, f16, 0)
            r0 = rows // 16 * 16

            def f1(q, _):
                pltpu.sync_copy(
                    wstage.at[pl.ds(pl.multiple_of((r0 + q) * 128, 8), 128)],
                    vox_out.at[pl.ds(pl.multiple_of(dst + (r0 + q) * 128, 8),
                                     128)])
                return 0

            lax.fori_loop(0, rows - r0, f1, 0)

        def unplace(i, _):
            p = s0 + i * 16 + lane
            m = p < e0
            ki = plsc.load_gather(ks_i, [jnp.minimum(p, CAP)], mask=m)
            sl = plsc.load_gather(ks_s, [ki], mask=m)
            off = jnp.where(m, (sl - w * 8192) * 4, 0)
            plsc.store_scatter(wstage, [off], zf16, mask=m)
            plsc.store_scatter(wstage, [off + 1], zf16, mask=m)
            plsc.store_scatter(wstage, [off + 2], zf16, mask=m)
            plsc.store_scatter(wstage, [off + 3], zf16, mask=m)
            return 0

        lax.fori_loop(0, (e0 - s0 + 15) // 16, unplace, 0)
        return 0

    lax.fori_loop(0, nwin, p4, 0)

    def p5(i, cw):
        c0 = i * 16 + lane
        h = hcell[pl.ds(i * 16, 16)]
        cnt = h & CNTMASK
        lr = lax.shift_right_logical(h, 18)
        ok = (cnt > 0) & (lr < nout)
        g = lo + c0
        yv = g // GX
        xv = g - yv * GX
        off = jnp.where(ok, (lr - cw * 256) * 8, 2168)
        plsc.store_scatter(cnst, [off], zeros16, mask=ok)
        plsc.store_scatter(cnst, [off + 1], yv, mask=ok)
        plsc.store_scatter(cnst, [off + 2], xv, mask=ok)
        plsc.store_scatter(cnst, [off + 3], jnp.minimum(cnt, MAXP), mask=ok)
        hi = jnp.max(jnp.where(ok, lr, 0))
        crossed = hi >= (cw + 1) * 256

        @pl.when(crossed)
        def _():
            pltpu.sync_copy(
                cnst.at[pl.ds(0, 2048)],
                cn_hbm.at[pl.ds(pl.multiple_of((rb + cw * 256) * 8, 8),
                                2048)])
            for q in range(8):
                cnst[pl.ds(q * 16, 16)] = cnst[pl.ds(2048 + q * 16, 16)]

        return jnp.where(crossed, cw + 1, cw)

    cw = lax.fori_loop(0, NCV, p5, jnp.int32(0))
    rem = jnp.maximum(nout - cw * 256, 0)

    def fr16(q, _):
        pltpu.sync_copy(
            cnst.at[pl.ds(pl.multiple_of(q * 128, 8), 128)],
            cn_hbm.at[pl.ds(pl.multiple_of((rb + cw * 256 + q * 16) * 8, 8),
                            128)])
        return 0

    lax.fori_loop(0, rem // 16, fr16, 0)
    rr0 = rem // 16 * 16

    def fr1(q, _):
        pltpu.sync_copy(
            cnst.at[pl.ds(pl.multiple_of((rr0 + q) * 8, 8), 8)],
            cn_hbm.at[pl.ds(pl.multiple_of((rb + cw * 256 + rr0 + q) * 8, 8),
                            8)])
        return 0

    lax.fori_loop(0, rem - rr0, fr1, 0)

    @pl.when(kept > CAP)
    def _():
        def clr(i, _):
            h = hcell[pl.ds(i * 16, 16)]
            hcell[pl.ds(i * 16, 16)] = h & ~CNTMASK
            return 0

        lax.fori_loop(0, NCV, clr, 0)

        def append_ovf(kc, keep, rel, x, y, z, w):
            pc2 = plsc.cumsum(keep.astype(jnp.int32))
            ovf = keep & ((kc + pc2 - 1) >= CAP)

            @pl.when(jnp.sum(ovf.astype(jnp.int32)) > 0)
            def _():
                base = (rb * 128) + rel * 4
                for comp, val in ((0, x), (1, y), (2, z), (3, w)):
                    ovi[...] = jnp.where(ovf, base + comp, VOXDUMPW)
                    ovv[...] = val
                    pltpu.async_copy(ovv, vox_out.at[ovi], sem).wait()

            return kc + jnp.sum(keep.astype(jnp.int32))

        emit_pass(jnp.int32(0), append_ovf)

    plsc.subcore_barrier()

    for c in range(30):
        @pl.when(t == c % NT)
        def _(c=c):
            nw = 2048 if c < 29 else 608
            w0 = c * 2048
            row0 = w0 // 3
            pltpu.sync_copy(cn_hbm.at[pl.ds(row0 * 8, 5504)],
                            rbuf.at[pl.ds(0, 5504)])

            def rp(j, _):
                wd = w0 + j * 16 + lane
                r = wd // 3
                src = (r - row0) * 8 + (wd - r * 3)
                v = plsc.load_gather(rbuf, [src])
                ostage[pl.ds(j * 16, 16)] = jnp.where(r < vn, v, -1)
                return 0

            lax.fori_loop(0, nw // 16, rp, 0)
            pltpu.sync_copy(ostage.at[pl.ds(0, nw)],
                            coor_out.at[pl.ds(w0, nw)])

    for c in range(20):
        @pl.when(t == c % NT)
        def _(c=c):
            nw = 1024 if c < 19 else 544
            w0 = c * 1024
            pltpu.sync_copy(cn_hbm.at[pl.ds(w0 * 8, 8192)], rbuf)

            def rp(j, _):
                wd = w0 + j * 16 + lane
                src = (wd - w0) * 8 + 3
                v = plsc.load_gather(rbuf, [src])
                ostage[pl.ds(j * 16, 16)] = jnp.where(wd < vn, v, 0)
                return 0

            lax.fori_loop(0, nw // 16, rp, 0)
            pltpu.sync_copy(ostage.at[pl.ds(0, nw)],
                            npv_out.at[pl.ds(w0, nw)])


def kernel(points):
    pts_flat = points.reshape(-1)
    vox, coor, npv, vnum = _vox_kernel(pts_flat)
    voxels = vox[: MAXV * MAXP * C].reshape(MAXV, MAXP, C)
    coors = coor[: MAXV * 3].reshape(MAXV, 3)
    return voxels, coors, npv[:MAXV], vnum[0]
